# Initial kernel scaffold; baseline (speedup 1.0000x reference)
import jax
import jax.numpy as jnp
from jax.experimental import pallas as pl


def kernel(x, W_sh_gate, W_sh_lin1, W_sh_lin2, W_router, W_down, W_up, W_e_gate, W_e_lin1, W_e_lin2):
    raise NotImplementedError("write your pallas kernel here")


# speedup vs baseline: 1.1138x; 1.1138x over previous
import functools

import jax
import jax.numpy as jnp
from jax.experimental import pallas as pl
from jax.experimental.pallas import tpu as pltpu

K = 8
SCALE = 2.5


def _dot_t(a, b):
    return jax.lax.dot_general(a, b, (((1,), (1,)), ((), ())),
                               preferred_element_type=jnp.float32)


def _sqrelu(v):
    return jnp.square(jnp.maximum(v, 0.0))


def _topk_weights(probs, k, scale):
    t, e = probs.shape
    eidx = jax.lax.broadcasted_iota(jnp.int32, (t, e), 1)
    rank = jnp.zeros((t, e), dtype=jnp.int32)
    for j in range(e):
        pj = probs[:, j:j + 1]
        beats = (pj > probs) | ((pj == probs) & (j < eidx))
        rank = rank + beats.astype(jnp.int32)
    w = jnp.where(rank < k, probs, 0.0)
    w = w / jnp.sum(w, axis=1, keepdims=True) * scale
    return w


def _pre_body(x_ref, wg_ref, wl1_ref, wl2_ref, wr_ref, wd_ref,
              sh_ref, lat_ref, w_ref):
    x = x_ref[...]
    g = _dot_t(x, wg_ref[...])
    h = _dot_t(x, wl1_ref[...])
    h = h * _sqrelu(g)
    sh_ref[...] = _dot_t(h, wl2_ref[...])
    lat_ref[...] = _dot_t(x, wd_ref[...])
    logits = _dot_t(x, wr_ref[...])
    probs = jax.nn.sigmoid(logits)
    w_ref[...] = _topk_weights(probs, K, SCALE)


def _moe_body(lat_ref, w_ref, sh_ref, wg_ref, wl1_ref, wl2_ref, wup_ref,
              out_ref, acc_ref):
    e = pl.program_id(1)
    ne = pl.num_programs(1)
    xl = lat_ref[...]
    g = _dot_t(xl, wg_ref[0])
    h = _dot_t(xl, wl1_ref[0])
    h = h * _sqrelu(g)
    oe = _dot_t(h, wl2_ref[0])
    wmat = w_ref[...]
    eidx = jax.lax.broadcasted_iota(jnp.int32, wmat.shape, 1)
    wcol = jnp.sum(jnp.where(eidx == e, wmat, 0.0), axis=1, keepdims=True)
    contrib = wcol * oe

    @pl.when(e == 0)
    def _():
        acc_ref[...] = contrib

    @pl.when(e > 0)
    def _():
        acc_ref[...] = acc_ref[...] + contrib

    @pl.when(e == ne - 1)
    def _():
        out_ref[...] = sh_ref[...] + _dot_t(acc_ref[...], wup_ref[...])


def kernel(x, W_sh_gate, W_sh_lin1, W_sh_lin2, W_router, W_down, W_up,
           W_e_gate, W_e_lin1, W_e_lin2):
    b, s, emb = x.shape
    hid = W_sh_gate.shape[0]
    lat = W_down.shape[0]
    ne = W_router.shape[0]
    t = b * s
    x2d = x.reshape(t, emb)
    tile = min(256, t)
    nt = t // tile

    shared, x_lat, wts = pl.pallas_call(
        _pre_body,
        grid=(nt,),
        in_specs=[
            pl.BlockSpec((tile, emb), lambda i: (i, 0)),
            pl.BlockSpec((hid, emb), lambda i: (0, 0)),
            pl.BlockSpec((hid, emb), lambda i: (0, 0)),
            pl.BlockSpec((emb, hid), lambda i: (0, 0)),
            pl.BlockSpec((ne, emb), lambda i: (0, 0)),
            pl.BlockSpec((lat, emb), lambda i: (0, 0)),
        ],
        out_specs=[
            pl.BlockSpec((tile, emb), lambda i: (i, 0)),
            pl.BlockSpec((tile, lat), lambda i: (i, 0)),
            pl.BlockSpec((tile, ne), lambda i: (i, 0)),
        ],
        out_shape=[
            jax.ShapeDtypeStruct((t, emb), jnp.float32),
            jax.ShapeDtypeStruct((t, lat), jnp.float32),
            jax.ShapeDtypeStruct((t, ne), jnp.float32),
        ],
    )(x2d, W_sh_gate, W_sh_lin1, W_sh_lin2, W_router, W_down)

    out = pl.pallas_call(
        _moe_body,
        grid=(nt, ne),
        in_specs=[
            pl.BlockSpec((tile, lat), lambda i, e: (i, 0)),
            pl.BlockSpec((tile, ne), lambda i, e: (i, 0)),
            pl.BlockSpec((tile, emb), lambda i, e: (i, 0)),
            pl.BlockSpec((1, hid, lat), lambda i, e: (e, 0, 0)),
            pl.BlockSpec((1, hid, lat), lambda i, e: (e, 0, 0)),
            pl.BlockSpec((1, lat, hid), lambda i, e: (e, 0, 0)),
            pl.BlockSpec((emb, lat), lambda i, e: (0, 0)),
        ],
        out_specs=pl.BlockSpec((tile, emb), lambda i, e: (i, 0)),
        out_shape=jax.ShapeDtypeStruct((t, emb), jnp.float32),
        scratch_shapes=[pltpu.VMEM((tile, lat), jnp.float32)],
    )(x_lat, wts, shared, W_e_gate, W_e_lin1, W_e_lin2, W_up)

    return out.reshape(b, s, emb)


# speedup vs baseline: 1.3777x; 1.2369x over previous
import jax
import jax.numpy as jnp
from jax.experimental import pallas as pl
from jax.experimental.pallas import tpu as pltpu

K = 8
SCALE = 2.5


def _dot_t(a, b):
    return jax.lax.dot_general(a, b, (((1,), (1,)), ((), ())),
                               preferred_element_type=jnp.float32)


def _sqrelu(v):
    return jnp.square(jnp.maximum(v, 0.0))


def _topk_weights(probs, k, scale):
    t, e = probs.shape
    eidx = jax.lax.broadcasted_iota(jnp.int32, (t, e), 1)
    rank = jnp.zeros((t, e), dtype=jnp.int32)
    for j in range(e):
        pj = probs[:, j:j + 1]
        beats = (pj > probs) | ((pj == probs) & (j < eidx))
        rank = rank + beats.astype(jnp.int32)
    w = jnp.where(rank < k, probs, 0.0)
    w = w / jnp.sum(w, axis=1, keepdims=True) * scale
    return w


def _pre_body(xf_ref, xb_ref, wg_ref, wl1_ref, wl2_ref, wr_ref, wd_ref,
              sh_ref, lat_ref, w_ref):
    xb = xb_ref[...]
    g = _dot_t(xb, wg_ref[...])
    h = _dot_t(xb, wl1_ref[...])
    h = (h * _sqrelu(g)).astype(jnp.bfloat16)
    sh_ref[...] = _dot_t(h, wl2_ref[...])
    lat_ref[...] = _dot_t(xb, wd_ref[...]).astype(jnp.bfloat16)
    logits = _dot_t(xf_ref[...], wr_ref[...])
    probs = jax.nn.sigmoid(logits)
    w_ref[...] = _topk_weights(probs, K, SCALE)


def _moe_body(lat_ref, w_ref, sh_ref, wg_ref, wl1_ref, wl2_ref, wup_ref,
              out_ref, acc_ref):
    eo = pl.program_id(0)
    neo = pl.num_programs(0)
    i = pl.program_id(1)
    epb = wg_ref.shape[0]
    tile = lat_ref.shape[0]
    xl = lat_ref[...]
    wmat = w_ref[...]
    eidx = jax.lax.broadcasted_iota(jnp.int32, wmat.shape, 1)

    acc = jnp.zeros((tile, wup_ref.shape[1]), dtype=jnp.float32)
    for j in range(epb):
        g = _dot_t(xl, wg_ref[j])
        h = _dot_t(xl, wl1_ref[j])
        h = (h * _sqrelu(g)).astype(jnp.bfloat16)
        oe = _dot_t(h, wl2_ref[j])
        e = eo * epb + j
        wcol = jnp.sum(jnp.where(eidx == e, wmat, 0.0), axis=1,
                       keepdims=True)
        acc = acc + wcol * oe

    row = pl.multiple_of(i * tile, tile)

    @pl.when(eo == 0)
    def _():
        acc_ref[pl.ds(row, tile), :] = acc

    @pl.when(eo > 0)
    def _():
        acc_ref[pl.ds(row, tile), :] = acc_ref[pl.ds(row, tile), :] + acc

    @pl.when(eo == neo - 1)
    def _():
        routed = (acc_ref[pl.ds(row, tile), :]).astype(jnp.bfloat16)
        out_ref[...] = sh_ref[...] + _dot_t(routed, wup_ref[...])


def kernel(x, W_sh_gate, W_sh_lin1, W_sh_lin2, W_router, W_down, W_up,
           W_e_gate, W_e_lin1, W_e_lin2):
    b, s, emb = x.shape
    hid = W_sh_gate.shape[0]
    lat = W_down.shape[0]
    ne = W_router.shape[0]
    t = b * s
    x2d = x.reshape(t, emb)
    xb = x2d.astype(jnp.bfloat16)
    bf = jnp.bfloat16
    tile = min(256, t)
    nt = t // tile
    neo = 4 if ne % 4 == 0 else 2
    epb = ne // neo

    shared, x_lat, wts = pl.pallas_call(
        _pre_body,
        grid=(nt,),
        in_specs=[
            pl.BlockSpec((tile, emb), lambda i: (i, 0)),
            pl.BlockSpec((tile, emb), lambda i: (i, 0)),
            pl.BlockSpec((hid, emb), lambda i: (0, 0)),
            pl.BlockSpec((hid, emb), lambda i: (0, 0)),
            pl.BlockSpec((emb, hid), lambda i: (0, 0)),
            pl.BlockSpec((ne, emb), lambda i: (0, 0)),
            pl.BlockSpec((lat, emb), lambda i: (0, 0)),
        ],
        out_specs=[
            pl.BlockSpec((tile, emb), lambda i: (i, 0)),
            pl.BlockSpec((tile, lat), lambda i: (i, 0)),
            pl.BlockSpec((tile, ne), lambda i: (i, 0)),
        ],
        out_shape=[
            jax.ShapeDtypeStruct((t, emb), jnp.float32),
            jax.ShapeDtypeStruct((t, lat), bf),
            jax.ShapeDtypeStruct((t, ne), jnp.float32),
        ],
    )(x2d, xb, W_sh_gate.astype(bf), W_sh_lin1.astype(bf),
      W_sh_lin2.astype(bf), W_router, W_down.astype(bf))

    out = pl.pallas_call(
        _moe_body,
        grid=(neo, nt),
        in_specs=[
            pl.BlockSpec((tile, lat), lambda eo, i: (i, 0)),
            pl.BlockSpec((tile, ne), lambda eo, i: (i, 0)),
            pl.BlockSpec((tile, emb), lambda eo, i: (i, 0)),
            pl.BlockSpec((epb, hid, lat), lambda eo, i: (eo, 0, 0)),
            pl.BlockSpec((epb, hid, lat), lambda eo, i: (eo, 0, 0)),
            pl.BlockSpec((epb, lat, hid), lambda eo, i: (eo, 0, 0)),
            pl.BlockSpec((emb, lat), lambda eo, i: (0, 0)),
        ],
        out_specs=pl.BlockSpec((tile, emb), lambda eo, i: (i, 0)),
        out_shape=jax.ShapeDtypeStruct((t, emb), jnp.float32),
        scratch_shapes=[pltpu.VMEM((t, lat), jnp.float32)],
    )(x_lat, wts, shared, W_e_gate.astype(bf), W_e_lin1.astype(bf),
      W_e_lin2.astype(bf), W_up.astype(bf))

    return out.reshape(b, s, emb)


# speedup vs baseline: 1.5557x; 1.1292x over previous
import jax
import jax.numpy as jnp
from jax.experimental import pallas as pl
from jax.experimental.pallas import tpu as pltpu

K = 8
SCALE = 2.5
BF = jnp.bfloat16


def _dot_t(a, b):
    return jax.lax.dot_general(a, b, (((1,), (1,)), ((), ())),
                               preferred_element_type=jnp.float32)


def _sqrelu(v):
    return jnp.square(jnp.maximum(v, 0.0))


def _topk_weights(probs, k, scale):
    t, e = probs.shape
    eidx = jax.lax.broadcasted_iota(jnp.int32, (t, e), 1)
    rank = jnp.zeros((t, e), dtype=jnp.int32)
    for j in range(e):
        pj = probs[:, j:j + 1]
        beats = (pj > probs) | ((pj == probs) & (j < eidx))
        rank = rank + beats.astype(jnp.int32)
    w = jnp.where(rank < k, probs, 0.0)
    w = w / jnp.sum(w, axis=1, keepdims=True) * scale
    return w


def _pre_body(x_ref, wg_ref, wl1_ref, wl2_ref, wr_ref, wd_ref,
              sh_ref, lat_ref, w_ref,
              wgb, wl1b, wl2b, wdb):
    i = pl.program_id(0)

    @pl.when(i == 0)
    def _():
        wgb[...] = wg_ref[...].astype(BF)
        wl1b[...] = wl1_ref[...].astype(BF)
        wl2b[...] = wl2_ref[...].astype(BF)
        wdb[...] = wd_ref[...].astype(BF)

    xf = x_ref[...]
    xb = xf.astype(BF)
    g = _dot_t(xb, wgb[...])
    h = _dot_t(xb, wl1b[...])
    h = (h * _sqrelu(g)).astype(BF)
    sh_ref[...] = _dot_t(h, wl2b[...])
    lat_ref[...] = _dot_t(xb, wdb[...]).astype(BF)
    logits = _dot_t(xf, wr_ref[...])
    probs = jax.nn.sigmoid(logits)
    w_ref[...] = _topk_weights(probs, K, SCALE)


def _moe_body(lat_ref, w_ref, wg_ref, wl1_ref, wl2_ref,
              routed_ref, acc_ref, wgb, wl1b, wl2b):
    eo = pl.program_id(0)
    neo = pl.num_programs(0)
    i = pl.program_id(1)
    epb = wg_ref.shape[0]
    tile = lat_ref.shape[0]
    lat = lat_ref.shape[1]

    @pl.when(i == 0)
    def _():
        wgb[...] = wg_ref[...].astype(BF)
        wl1b[...] = wl1_ref[...].astype(BF)
        wl2b[...] = wl2_ref[...].astype(BF)

    xl = lat_ref[...]
    wmat = w_ref[...]
    eidx = jax.lax.broadcasted_iota(jnp.int32, wmat.shape, 1)

    acc = jnp.zeros((tile, lat), dtype=jnp.float32)
    for j in range(epb):
        g = _dot_t(xl, wgb[j])
        h = _dot_t(xl, wl1b[j])
        h = (h * _sqrelu(g)).astype(BF)
        oe = _dot_t(h, wl2b[j])
        e = eo * epb + j
        wcol = jnp.sum(jnp.where(eidx == e, wmat, 0.0), axis=1,
                       keepdims=True)
        acc = acc + wcol * oe

    row = pl.multiple_of(i * tile, tile)

    @pl.when(eo == 0)
    def _():
        acc_ref[pl.ds(row, tile), :] = acc

    @pl.when(eo > 0)
    def _():
        acc_ref[pl.ds(row, tile), :] = acc_ref[pl.ds(row, tile), :] + acc

    @pl.when(eo == neo - 1)
    def _():
        routed_ref[...] = acc_ref[pl.ds(row, tile), :].astype(BF)


def _post_body(routed_ref, sh_ref, wup_ref, out_ref, wupb):
    i = pl.program_id(0)

    @pl.when(i == 0)
    def _():
        wupb[...] = wup_ref[...].astype(BF)

    out_ref[...] = sh_ref[...] + _dot_t(routed_ref[...], wupb[...])


def kernel(x, W_sh_gate, W_sh_lin1, W_sh_lin2, W_router, W_down, W_up,
           W_e_gate, W_e_lin1, W_e_lin2):
    b, s, emb = x.shape
    hid = W_sh_gate.shape[0]
    lat = W_down.shape[0]
    ne = W_router.shape[0]
    t = b * s
    x2d = x.reshape(t, emb)
    tile = min(256, t)
    nt = t // tile
    neo = 8 if ne % 8 == 0 else ne
    epb = ne // neo

    shared, x_lat, wts = pl.pallas_call(
        _pre_body,
        grid=(nt,),
        in_specs=[
            pl.BlockSpec((tile, emb), lambda i: (i, 0)),
            pl.BlockSpec((hid, emb), lambda i: (0, 0)),
            pl.BlockSpec((hid, emb), lambda i: (0, 0)),
            pl.BlockSpec((emb, hid), lambda i: (0, 0)),
            pl.BlockSpec((ne, emb), lambda i: (0, 0)),
            pl.BlockSpec((lat, emb), lambda i: (0, 0)),
        ],
        out_specs=[
            pl.BlockSpec((tile, emb), lambda i: (i, 0)),
            pl.BlockSpec((tile, lat), lambda i: (i, 0)),
            pl.BlockSpec((tile, ne), lambda i: (i, 0)),
        ],
        out_shape=[
            jax.ShapeDtypeStruct((t, emb), jnp.float32),
            jax.ShapeDtypeStruct((t, lat), BF),
            jax.ShapeDtypeStruct((t, ne), jnp.float32),
        ],
        scratch_shapes=[
            pltpu.VMEM((hid, emb), BF),
            pltpu.VMEM((hid, emb), BF),
            pltpu.VMEM((emb, hid), BF),
            pltpu.VMEM((lat, emb), BF),
        ],
    )(x2d, W_sh_gate, W_sh_lin1, W_sh_lin2, W_router, W_down)

    routed = pl.pallas_call(
        _moe_body,
        grid=(neo, nt),
        in_specs=[
            pl.BlockSpec((tile, lat), lambda eo, i: (i, 0)),
            pl.BlockSpec((tile, ne), lambda eo, i: (i, 0)),
            pl.BlockSpec((epb, hid, lat), lambda eo, i: (eo, 0, 0)),
            pl.BlockSpec((epb, hid, lat), lambda eo, i: (eo, 0, 0)),
            pl.BlockSpec((epb, lat, hid), lambda eo, i: (eo, 0, 0)),
        ],
        out_specs=pl.BlockSpec((tile, lat), lambda eo, i: (i, 0)),
        out_shape=jax.ShapeDtypeStruct((t, lat), BF),
        scratch_shapes=[
            pltpu.VMEM((t, lat), jnp.float32),
            pltpu.VMEM((epb, hid, lat), BF),
            pltpu.VMEM((epb, hid, lat), BF),
            pltpu.VMEM((epb, lat, hid), BF),
        ],
    )(x_lat, wts, W_e_gate, W_e_lin1, W_e_lin2)

    out = pl.pallas_call(
        _post_body,
        grid=(nt,),
        in_specs=[
            pl.BlockSpec((tile, lat), lambda i: (i, 0)),
            pl.BlockSpec((tile, emb), lambda i: (i, 0)),
            pl.BlockSpec((emb, lat), lambda i: (0, 0)),
        ],
        out_specs=pl.BlockSpec((tile, emb), lambda i: (i, 0)),
        out_shape=jax.ShapeDtypeStruct((t, emb), jnp.float32),
        scratch_shapes=[pltpu.VMEM((emb, lat), BF)],
    )(routed, shared, W_up)

    return out.reshape(b, s, emb)


# speedup vs baseline: 1.7342x; 1.1147x over previous
import jax
import jax.numpy as jnp
from jax.experimental import pallas as pl
from jax.experimental.pallas import tpu as pltpu

K = 8
SCALE = 2.5
BF = jnp.bfloat16


def _dot_t(a, b):
    return jax.lax.dot_general(a, b, (((1,), (1,)), ((), ())),
                               preferred_element_type=jnp.float32)


def _sqrelu(v):
    return jnp.square(jnp.maximum(v, 0.0))


def _topk_weights(probs, k, scale):
    t, e = probs.shape
    eidx = jax.lax.broadcasted_iota(jnp.int32, (t, e), 1)
    rank = jnp.zeros((t, e), dtype=jnp.int32)
    for j in range(e):
        pj = probs[:, j:j + 1]
        beats = (pj > probs) | ((pj == probs) & (j < eidx))
        rank = rank + beats.astype(jnp.int32)
    w = jnp.where(rank < k, probs, 0.0)
    w = w / jnp.sum(w, axis=1, keepdims=True) * scale
    return w


def _pre_body(x_ref, wg_ref, wl1_ref, wl2_ref, wr_ref, wd_ref,
              sh_ref, lat_ref, w_ref,
              wgb, wl1b, wl2b, wdb):
    i = pl.program_id(0)

    @pl.when(i == 0)
    def _():
        wgb[...] = wg_ref[...].astype(BF)
        wl1b[...] = wl1_ref[...].astype(BF)
        wl2b[...] = wl2_ref[...].astype(BF)
        wdb[...] = wd_ref[...].astype(BF)

    xf = x_ref[...]
    xb = xf.astype(BF)
    g = _dot_t(xb, wgb[...])
    h = _dot_t(xb, wl1b[...])
    h = (h * _sqrelu(g)).astype(BF)
    sh_ref[...] = _dot_t(h, wl2b[...])
    lat_ref[...] = _dot_t(xb, wdb[...]).astype(BF)
    logits = _dot_t(xf, wr_ref[...])
    probs = jax.nn.sigmoid(logits)
    w_ref[...] = _topk_weights(probs, K, SCALE)


def _moe_body(lat_ref, w_ref, wg_ref, wl1_ref, wl2_ref,
              routed_ref, acc_ref, wgb, wl1b, wl2b):
    eo = pl.program_id(0)
    neo = pl.num_programs(0)
    i = pl.program_id(1)
    epb = wg_ref.shape[0]
    tile = lat_ref.shape[0]
    lat = lat_ref.shape[1]

    @pl.when(i == 0)
    def _():
        wgb[...] = wg_ref[...].astype(BF)
        wl1b[...] = wl1_ref[...].astype(BF)
        wl2b[...] = wl2_ref[...].astype(BF)

    xl = lat_ref[...]
    wmat = w_ref[...]
    eidx = jax.lax.broadcasted_iota(jnp.int32, wmat.shape, 1)

    acc = jnp.zeros((tile, lat), dtype=jnp.float32)
    for j in range(epb):
        g = _dot_t(xl, wgb[j])
        h = _dot_t(xl, wl1b[j])
        h = (h * _sqrelu(g)).astype(BF)
        oe = _dot_t(h, wl2b[j])
        e = eo * epb + j
        wcol = jnp.sum(jnp.where(eidx == e, wmat, 0.0), axis=1,
                       keepdims=True)
        acc = acc + wcol * oe

    row = pl.multiple_of(i * tile, tile)

    @pl.when(eo == 0)
    def _():
        acc_ref[pl.ds(row, tile), :] = acc

    @pl.when(eo > 0)
    def _():
        acc_ref[pl.ds(row, tile), :] = acc_ref[pl.ds(row, tile), :] + acc

    @pl.when(eo == neo - 1)
    def _():
        routed_ref[...] = acc_ref[pl.ds(row, tile), :].astype(BF)


def _post_body(routed_ref, sh_ref, wup_ref, out_ref, wupb):
    i = pl.program_id(0)

    @pl.when(i == 0)
    def _():
        wupb[...] = wup_ref[...].astype(BF)

    out_ref[...] = sh_ref[...] + _dot_t(routed_ref[...], wupb[...])


def kernel(x, W_sh_gate, W_sh_lin1, W_sh_lin2, W_router, W_down, W_up,
           W_e_gate, W_e_lin1, W_e_lin2):
    b, s, emb = x.shape
    hid = W_sh_gate.shape[0]
    lat = W_down.shape[0]
    ne = W_router.shape[0]
    t = b * s
    x2d = x.reshape(t, emb)
    tile = min(256, t)
    nt = t // tile
    neo = 8 if ne % 8 == 0 else ne
    epb = ne // neo

    shared, x_lat, wts = pl.pallas_call(
        _pre_body,
        grid=(nt,),
        in_specs=[
            pl.BlockSpec((tile, emb), lambda i: (i, 0)),
            pl.BlockSpec((hid, emb), lambda i: (0, 0)),
            pl.BlockSpec((hid, emb), lambda i: (0, 0)),
            pl.BlockSpec((emb, hid), lambda i: (0, 0)),
            pl.BlockSpec((ne, emb), lambda i: (0, 0)),
            pl.BlockSpec((lat, emb), lambda i: (0, 0)),
        ],
        out_specs=[
            pl.BlockSpec((tile, emb), lambda i: (i, 0)),
            pl.BlockSpec((tile, lat), lambda i: (i, 0)),
            pl.BlockSpec((tile, ne), lambda i: (i, 0)),
        ],
        out_shape=[
            jax.ShapeDtypeStruct((t, emb), jnp.float32),
            jax.ShapeDtypeStruct((t, lat), BF),
            jax.ShapeDtypeStruct((t, ne), jnp.float32),
        ],
        scratch_shapes=[
            pltpu.VMEM((hid, emb), BF),
            pltpu.VMEM((hid, emb), BF),
            pltpu.VMEM((emb, hid), BF),
            pltpu.VMEM((lat, emb), BF),
        ],
    )(x2d, W_sh_gate, W_sh_lin1, W_sh_lin2, W_router, W_down)

    mtile = min(512, t)
    nmt = t // mtile
    routed = pl.pallas_call(
        _moe_body,
        grid=(neo, nmt),
        in_specs=[
            pl.BlockSpec((mtile, lat), lambda eo, i: (i, 0)),
            pl.BlockSpec((mtile, ne), lambda eo, i: (i, 0)),
            pl.BlockSpec((epb, hid, lat), lambda eo, i: (eo, 0, 0)),
            pl.BlockSpec((epb, hid, lat), lambda eo, i: (eo, 0, 0)),
            pl.BlockSpec((epb, lat, hid), lambda eo, i: (eo, 0, 0)),
        ],
        out_specs=pl.BlockSpec((mtile, lat), lambda eo, i: (i, 0)),
        out_shape=jax.ShapeDtypeStruct((t, lat), BF),
        scratch_shapes=[
            pltpu.VMEM((t, lat), jnp.float32),
            pltpu.VMEM((epb, hid, lat), BF),
            pltpu.VMEM((epb, hid, lat), BF),
            pltpu.VMEM((epb, lat, hid), BF),
        ],
    )(x_lat, wts, W_e_gate, W_e_lin1, W_e_lin2)

    out = pl.pallas_call(
        _post_body,
        grid=(nt,),
        in_specs=[
            pl.BlockSpec((tile, lat), lambda i: (i, 0)),
            pl.BlockSpec((tile, emb), lambda i: (i, 0)),
            pl.BlockSpec((emb, lat), lambda i: (0, 0)),
        ],
        out_specs=pl.BlockSpec((tile, emb), lambda i: (i, 0)),
        out_shape=jax.ShapeDtypeStruct((t, emb), jnp.float32),
        scratch_shapes=[pltpu.VMEM((emb, lat), BF)],
    )(routed, shared, W_up)

    return out.reshape(b, s, emb)


# speedup vs baseline: 1.8697x; 1.0781x over previous
import jax
import jax.numpy as jnp
from jax.experimental import pallas as pl
from jax.experimental.pallas import tpu as pltpu

K = 8
SCALE = 2.5
BF = jnp.bfloat16


def _dot_t(a, b):
    return jax.lax.dot_general(a, b, (((1,), (1,)), ((), ())),
                               preferred_element_type=jnp.float32)


def _sqrelu(v):
    return jnp.square(jnp.maximum(v, 0.0))


def _topk_weights(probs, k, scale):
    t, e = probs.shape
    eidx = jax.lax.broadcasted_iota(jnp.int32, (t, e), 1)
    rank = jnp.zeros((t, e), dtype=jnp.int32)
    for j in range(e):
        pj = probs[:, j:j + 1]
        beats = (pj > probs) | ((pj == probs) & (j < eidx))
        rank = rank + beats.astype(jnp.int32)
    w = jnp.where(rank < k, probs, 0.0)
    w = w / jnp.sum(w, axis=1, keepdims=True) * scale
    return w


def _pre_body(x_ref, wg_ref, wl1_ref, wl2_ref, wr_ref, wd_ref,
              sh_ref, lat_ref, w_ref,
              wgb, wl1b, wl2b, wdb):
    i = pl.program_id(0)

    @pl.when(i == 0)
    def _():
        wgb[...] = wg_ref[...].astype(BF)
        wl1b[...] = wl1_ref[...].astype(BF)
        wl2b[...] = wl2_ref[...].astype(BF)
        wdb[...] = wd_ref[...].astype(BF)

    xf = x_ref[...]
    xb = xf.astype(BF)
    g = _dot_t(xb, wgb[...])
    h = _dot_t(xb, wl1b[...])
    h = (h * _sqrelu(g)).astype(BF)
    sh_ref[...] = _dot_t(h, wl2b[...])
    lat_ref[...] = _dot_t(xb, wdb[...]).astype(BF)
    logits = _dot_t(xf, wr_ref[...])
    probs = jax.nn.sigmoid(logits)
    w_ref[...] = _topk_weights(probs, K, SCALE)


def _moe_body(lat_ref, w_ref, wg_ref, wl1_ref, wl2_ref,
              routed_ref, acc_ref, wgb, wl1b, wl2b):
    eo = pl.program_id(0)
    neo = pl.num_programs(0)
    i = pl.program_id(1)
    epb = wg_ref.shape[0]
    tile = lat_ref.shape[0]
    lat = lat_ref.shape[1]

    @pl.when(i == 0)
    def _():
        wgb[...] = wg_ref[...].astype(BF)
        wl1b[...] = wl1_ref[...].astype(BF)
        wl2b[...] = wl2_ref[...].astype(BF)

    xl = lat_ref[...]
    wmat = w_ref[...]
    eidx = jax.lax.broadcasted_iota(jnp.int32, wmat.shape, 1)

    acc = jnp.zeros((tile, lat), dtype=jnp.float32)
    for j in range(epb):
        g = _dot_t(xl, wgb[j])
        h = _dot_t(xl, wl1b[j])
        h = (h * _sqrelu(g)).astype(BF)
        oe = _dot_t(h, wl2b[j])
        e = eo * epb + j
        wcol = jnp.sum(jnp.where(eidx == e, wmat, 0.0), axis=1,
                       keepdims=True)
        acc = acc + wcol * oe

    row = pl.multiple_of(i * tile, tile)

    @pl.when(eo == 0)
    def _():
        acc_ref[pl.ds(row, tile), :] = acc

    @pl.when(eo > 0)
    def _():
        acc_ref[pl.ds(row, tile), :] = acc_ref[pl.ds(row, tile), :] + acc

    @pl.when(eo == neo - 1)
    def _():
        routed_ref[...] = acc_ref[pl.ds(row, tile), :].astype(BF)


def _post_body(routed_ref, sh_ref, wup_ref, out_ref, wupb):
    i = pl.program_id(0)

    @pl.when(i == 0)
    def _():
        wupb[...] = wup_ref[...].astype(BF)

    out_ref[...] = sh_ref[...] + _dot_t(routed_ref[...], wupb[...])


def kernel(x, W_sh_gate, W_sh_lin1, W_sh_lin2, W_router, W_down, W_up,
           W_e_gate, W_e_lin1, W_e_lin2):
    b, s, emb = x.shape
    hid = W_sh_gate.shape[0]
    lat = W_down.shape[0]
    ne = W_router.shape[0]
    t = b * s
    x2d = x.reshape(t, emb)
    tile = min(256, t)
    nt = t // tile
    neo = 8 if ne % 8 == 0 else ne
    epb = ne // neo

    shared, x_lat, wts = pl.pallas_call(
        _pre_body,
        grid=(nt,),
        in_specs=[
            pl.BlockSpec((tile, emb), lambda i: (i, 0)),
            pl.BlockSpec((hid, emb), lambda i: (0, 0)),
            pl.BlockSpec((hid, emb), lambda i: (0, 0)),
            pl.BlockSpec((emb, hid), lambda i: (0, 0)),
            pl.BlockSpec((ne, emb), lambda i: (0, 0)),
            pl.BlockSpec((lat, emb), lambda i: (0, 0)),
        ],
        out_specs=[
            pl.BlockSpec((tile, emb), lambda i: (i, 0)),
            pl.BlockSpec((tile, lat), lambda i: (i, 0)),
            pl.BlockSpec((tile, ne), lambda i: (i, 0)),
        ],
        out_shape=[
            jax.ShapeDtypeStruct((t, emb), jnp.float32),
            jax.ShapeDtypeStruct((t, lat), BF),
            jax.ShapeDtypeStruct((t, ne), jnp.float32),
        ],
        scratch_shapes=[
            pltpu.VMEM((hid, emb), BF),
            pltpu.VMEM((hid, emb), BF),
            pltpu.VMEM((emb, hid), BF),
            pltpu.VMEM((lat, emb), BF),
        ],
    )(x2d, W_sh_gate, W_sh_lin1, W_sh_lin2, W_router, W_down)

    mtile = min(1024, t)
    nmt = t // mtile
    routed = pl.pallas_call(
        _moe_body,
        grid=(neo, nmt),
        in_specs=[
            pl.BlockSpec((mtile, lat), lambda eo, i: (i, 0)),
            pl.BlockSpec((mtile, ne), lambda eo, i: (i, 0)),
            pl.BlockSpec((epb, hid, lat), lambda eo, i: (eo, 0, 0)),
            pl.BlockSpec((epb, hid, lat), lambda eo, i: (eo, 0, 0)),
            pl.BlockSpec((epb, lat, hid), lambda eo, i: (eo, 0, 0)),
        ],
        out_specs=pl.BlockSpec((mtile, lat), lambda eo, i: (i, 0)),
        out_shape=jax.ShapeDtypeStruct((t, lat), BF),
        scratch_shapes=[
            pltpu.VMEM((t, lat), jnp.float32),
            pltpu.VMEM((epb, hid, lat), BF),
            pltpu.VMEM((epb, hid, lat), BF),
            pltpu.VMEM((epb, lat, hid), BF),
        ],
    )(x_lat, wts, W_e_gate, W_e_lin1, W_e_lin2)

    out = pl.pallas_call(
        _post_body,
        grid=(nt,),
        in_specs=[
            pl.BlockSpec((tile, lat), lambda i: (i, 0)),
            pl.BlockSpec((tile, emb), lambda i: (i, 0)),
            pl.BlockSpec((emb, lat), lambda i: (0, 0)),
        ],
        out_specs=pl.BlockSpec((tile, emb), lambda i: (i, 0)),
        out_shape=jax.ShapeDtypeStruct((t, emb), jnp.float32),
        scratch_shapes=[pltpu.VMEM((emb, lat), BF)],
    )(routed, shared, W_up)

    return out.reshape(b, s, emb)


# speedup vs baseline: 1.8921x; 1.0120x over previous
import jax
import jax.numpy as jnp
from jax.experimental import pallas as pl
from jax.experimental.pallas import tpu as pltpu

K = 8
SCALE = 2.5
BF = jnp.bfloat16


def _dot_t(a, b):
    return jax.lax.dot_general(a, b, (((1,), (1,)), ((), ())),
                               preferred_element_type=jnp.float32)


def _sqrelu(v):
    return jnp.square(jnp.maximum(v, 0.0))


def _topk_weights(probs, k, scale):
    t, e = probs.shape
    eidx = jax.lax.broadcasted_iota(jnp.int32, (t, e), 1)
    rank = jnp.zeros((t, e), dtype=jnp.int32)
    for j in range(e):
        pj = probs[:, j:j + 1]
        beats = (pj > probs) | ((pj == probs) & (j < eidx))
        rank = rank + beats.astype(jnp.int32)
    w = jnp.where(rank < k, probs, 0.0)
    w = w / jnp.sum(w, axis=1, keepdims=True) * scale
    return w


def _pre_body(x_ref, wg_ref, wl1_ref, wl2_ref, wr_ref, wd_ref,
              sh_ref, lat_ref, w_ref,
              wgb, wl1b, wl2b, wdb):
    i = pl.program_id(0)

    @pl.when(i == 0)
    def _():
        wgb[...] = wg_ref[...].astype(BF)
        wl1b[...] = wl1_ref[...].astype(BF)
        wl2b[...] = wl2_ref[...].astype(BF)
        wdb[...] = wd_ref[...].astype(BF)

    xf = x_ref[...]
    xb = xf.astype(BF)
    g = _dot_t(xb, wgb[...])
    h = _dot_t(xb, wl1b[...])
    h = (h * _sqrelu(g)).astype(BF)
    sh_ref[...] = _dot_t(h, wl2b[...])
    lat_ref[...] = _dot_t(xb, wdb[...]).astype(BF)
    logits = _dot_t(xf, wr_ref[...])
    probs = jax.nn.sigmoid(logits)
    w_ref[...] = _topk_weights(probs, K, SCALE)


def _moe_body(lat_ref, w_ref, wg_ref, wl1_ref, wl2_ref,
              routed_ref, acc_ref, wgb, wl1b, wl2b):
    eo = pl.program_id(0)
    neo = pl.num_programs(0)
    i = pl.program_id(1)
    epb = wg_ref.shape[0]
    tile = lat_ref.shape[0]
    lat = lat_ref.shape[1]

    @pl.when(i == 0)
    def _():
        wgb[...] = wg_ref[...].astype(BF)
        wl1b[...] = wl1_ref[...].astype(BF)
        wl2b[...] = wl2_ref[...].astype(BF)

    xl = lat_ref[...]
    wmat = w_ref[...]
    eidx = jax.lax.broadcasted_iota(jnp.int32, wmat.shape, 1)

    acc = jnp.zeros((tile, lat), dtype=jnp.float32)
    for j in range(epb):
        g = _dot_t(xl, wgb[j])
        h = _dot_t(xl, wl1b[j])
        h = (h * _sqrelu(g)).astype(BF)
        oe = _dot_t(h, wl2b[j])
        e = eo * epb + j
        wcol = jnp.sum(jnp.where(eidx == e, wmat, 0.0), axis=1,
                       keepdims=True)
        acc = acc + wcol * oe

    row = pl.multiple_of(i * tile, tile)

    @pl.when(eo == 0)
    def _():
        acc_ref[pl.ds(row, tile), :] = acc

    @pl.when(eo > 0)
    def _():
        acc_ref[pl.ds(row, tile), :] = acc_ref[pl.ds(row, tile), :] + acc

    @pl.when(eo == neo - 1)
    def _():
        routed_ref[...] = acc_ref[pl.ds(row, tile), :].astype(BF)


def _post_body(routed_ref, sh_ref, wup_ref, out_ref, wupb):
    i = pl.program_id(0)

    @pl.when(i == 0)
    def _():
        wupb[...] = wup_ref[...].astype(BF)

    out_ref[...] = sh_ref[...] + _dot_t(routed_ref[...], wupb[...])


def kernel(x, W_sh_gate, W_sh_lin1, W_sh_lin2, W_router, W_down, W_up,
           W_e_gate, W_e_lin1, W_e_lin2):
    b, s, emb = x.shape
    hid = W_sh_gate.shape[0]
    lat = W_down.shape[0]
    ne = W_router.shape[0]
    t = b * s
    x2d = x.reshape(t, emb)
    tile = min(256, t)
    nt = t // tile
    neo = 8 if ne % 8 == 0 else ne
    epb = ne // neo

    shared, x_lat, wts = pl.pallas_call(
        _pre_body,
        grid=(nt,),
        in_specs=[
            pl.BlockSpec((tile, emb), lambda i: (i, 0)),
            pl.BlockSpec((hid, emb), lambda i: (0, 0)),
            pl.BlockSpec((hid, emb), lambda i: (0, 0)),
            pl.BlockSpec((emb, hid), lambda i: (0, 0)),
            pl.BlockSpec((ne, emb), lambda i: (0, 0)),
            pl.BlockSpec((lat, emb), lambda i: (0, 0)),
        ],
        out_specs=[
            pl.BlockSpec((tile, emb), lambda i: (i, 0)),
            pl.BlockSpec((tile, lat), lambda i: (i, 0)),
            pl.BlockSpec((tile, ne), lambda i: (i, 0)),
        ],
        out_shape=[
            jax.ShapeDtypeStruct((t, emb), jnp.float32),
            jax.ShapeDtypeStruct((t, lat), BF),
            jax.ShapeDtypeStruct((t, ne), jnp.float32),
        ],
        scratch_shapes=[
            pltpu.VMEM((hid, emb), BF),
            pltpu.VMEM((hid, emb), BF),
            pltpu.VMEM((emb, hid), BF),
            pltpu.VMEM((lat, emb), BF),
        ],
    )(x2d, W_sh_gate, W_sh_lin1, W_sh_lin2, W_router, W_down)

    mtile = min(2048, t)
    nmt = t // mtile
    routed = pl.pallas_call(
        _moe_body,
        grid=(neo, nmt),
        in_specs=[
            pl.BlockSpec((mtile, lat), lambda eo, i: (i, 0)),
            pl.BlockSpec((mtile, ne), lambda eo, i: (i, 0)),
            pl.BlockSpec((epb, hid, lat), lambda eo, i: (eo, 0, 0)),
            pl.BlockSpec((epb, hid, lat), lambda eo, i: (eo, 0, 0)),
            pl.BlockSpec((epb, lat, hid), lambda eo, i: (eo, 0, 0)),
        ],
        out_specs=pl.BlockSpec((mtile, lat), lambda eo, i: (i, 0)),
        out_shape=jax.ShapeDtypeStruct((t, lat), BF),
        scratch_shapes=[
            pltpu.VMEM((t, lat), jnp.float32),
            pltpu.VMEM((epb, hid, lat), BF),
            pltpu.VMEM((epb, hid, lat), BF),
            pltpu.VMEM((epb, lat, hid), BF),
        ],
    )(x_lat, wts, W_e_gate, W_e_lin1, W_e_lin2)

    out = pl.pallas_call(
        _post_body,
        grid=(nt,),
        in_specs=[
            pl.BlockSpec((tile, lat), lambda i: (i, 0)),
            pl.BlockSpec((tile, emb), lambda i: (i, 0)),
            pl.BlockSpec((emb, lat), lambda i: (0, 0)),
        ],
        out_specs=pl.BlockSpec((tile, emb), lambda i: (i, 0)),
        out_shape=jax.ShapeDtypeStruct((t, emb), jnp.float32),
        scratch_shapes=[pltpu.VMEM((emb, lat), BF)],
    )(routed, shared, W_up)

    return out.reshape(b, s, emb)
